# baseline (speedup 1.0000x reference)
import jax
import jax.numpy as jnp
from jax import lax
from jax.experimental import pallas as pl
from jax.experimental.pallas import tpu as pltpu
from jax.experimental.pallas import tpu_sc as plsc

_B = 1024
_C = 19
_NC = 2
_NW = 32
_TBL = 2 * _C * _B
_R = 4


def _sc_body(x_hbm, t_hbm, out_hbm, xs0, ts0, xs1, ts1, sem0, sem1, table):
    n_img, c_img, h_img, w_img = x_hbm.shape
    nvec = (_R * w_img) // 16
    kpr = w_img // 16
    hb_per_img = h_img // _R
    per_w = (n_img * hb_per_img) // _NW
    nhalf = per_w // 2

    wid = lax.axis_index("s") * _NC + lax.axis_index("c")

    def zero(i, carry):
        table[pl.ds(i * 16, 16)] = jnp.zeros((16,), jnp.float32)
        return carry

    lax.fori_loop(0, _TBL // 16, zero, 0)

    ones = jnp.ones((16,), jnp.float32)

    def start_chunk(cid, xs, ts, sem):
        n = cid // hb_per_img
        h0 = (cid % hb_per_img) * _R
        for c in range(_C):
            pltpu.async_copy(x_hbm.at[n, c, pl.ds(h0, _R), :], xs.at[c], sem)
        pltpu.async_copy(t_hbm.at[n, pl.ds(h0, _R), :], ts, sem)

    def drain_chunk(xs, ts, sem):
        for c in range(_C):
            pltpu.make_async_copy(
                x_hbm.at[0, c, pl.ds(0, _R), :], xs.at[c], sem).wait()
        pltpu.make_async_copy(t_hbm.at[0, pl.ds(0, _R), :], ts, sem).wait()

    def process(xs, ts):
        lanes = lax.iota(jnp.int32, 16)

        @plsc.parallel_loop(0, nvec, 1, unroll=1)
        def vb(j):
            r = j // kpr
            k = (j % kpr) * 16
            t = ts[r, pl.ds(k, 16)]
            es = []
            s = None
            for c in range(_C):
                e = jnp.exp(xs[c, r, pl.ds(k, 16)])
                es.append(e)
                s = e if s is None else s + e
            rs = 1.0 / s
            for c in range(_C):
                p = es[c] * rs
                isfg = t == c
                e = jnp.where(isfg, 1.0 - p, p)
                b = jnp.minimum((e * _B).astype(jnp.int32), _B - 1)
                plsc.addupdate_scatter(table, [b + (c * _B)], ones)
            xt = plsc.load_gather(xs, [t, jnp.full((16,), r, jnp.int32),
                                       k + lanes])
            pt = jnp.exp(xt) * rs
            bfg = jnp.minimum(((1.0 - pt) * _B).astype(jnp.int32), _B - 1)
            plsc.addupdate_scatter(table, [bfg + t * _B + (_C * _B)], ones)

    base = wid * per_w
    start_chunk(base, xs0, ts0, sem0)

    def pair(i, carry):
        c0 = base + 2 * i
        start_chunk(c0 + 1, xs1, ts1, sem1)
        drain_chunk(xs0, ts0, sem0)
        process(xs0, ts0)

        @pl.when(i + 1 < nhalf)
        def _():
            start_chunk(c0 + 2, xs0, ts0, sem0)

        drain_chunk(xs1, ts1, sem1)
        process(xs1, ts1)
        return carry

    lax.fori_loop(0, nhalf, pair, 0)
    pltpu.sync_copy(table, out_hbm.at[wid])


def _histogram(x4, t3):
    mesh = plsc.VectorSubcoreMesh(core_axis_name="c", subcore_axis_name="s")
    return pl.kernel(
        _sc_body,
        out_type=jax.ShapeDtypeStruct((_NW, _TBL), jnp.float32),
        mesh=mesh,
        scratch_types=[
            pltpu.VMEM((_C, _R, x4.shape[3]), jnp.float32),
            pltpu.VMEM((_R, x4.shape[3]), jnp.int32),
            pltpu.VMEM((_C, _R, x4.shape[3]), jnp.float32),
            pltpu.VMEM((_R, x4.shape[3]), jnp.int32),
            pltpu.SemaphoreType.DMA,
            pltpu.SemaphoreType.DMA,
            pltpu.VMEM((_TBL,), jnp.float32),
        ],
        compiler_params=pltpu.CompilerParams(needs_layout_passes=False),
    )(x4, t3)


def _post_body(h_ref, tri_ref, out_ref):
    h = jnp.sum(h_ref[...], axis=0)
    cnt = h[:_C]
    fgc = h[_C:]
    tri = tri_ref[...]
    r = lax.dot(cnt, tri, precision=lax.Precision.HIGHEST)
    f = lax.dot(fgc, tri, precision=lax.Precision.HIGHEST)
    g = f[:, 0:1]
    u = g + r - f
    jac = 1.0 - (g - f) / jnp.maximum(u, 1.0)
    jnext = jnp.concatenate([jac[:, 1:], jnp.zeros((_C, 1), jnp.float32)],
                            axis=1)
    dj = jac - jnext
    mid = ((lax.broadcasted_iota(jnp.int32, (_C, _B), 1)
            .astype(jnp.float32)) + 0.5) / _B
    losses = jnp.sum(mid * dj, axis=1)
    present = (g[:, 0] > 0.0).astype(jnp.float32)
    loss = jnp.sum(losses * present) / jnp.maximum(jnp.sum(present), 1.0)
    out_ref[...] = jnp.reshape(loss, (1, 1))


def _post(hist3, tri):
    return pl.pallas_call(
        _post_body,
        out_shape=jax.ShapeDtypeStruct((1, 1), jnp.float32),
    )(hist3, tri)


def kernel(input, target):
    hist = _histogram(input, target.astype(jnp.int32))
    hist3 = hist.reshape(_NW, 2 * _C, _B)
    tri = jnp.tri(_B, dtype=jnp.float32)
    out = _post(hist3, tri)
    return out[0, 0]


# speedup vs baseline: 1.3041x; 1.3041x over previous
import jax
import jax.numpy as jnp
from jax import lax
from jax.experimental import pallas as pl
from jax.experimental.pallas import tpu as pltpu
from jax.experimental.pallas import tpu_sc as plsc

_B = 1024
_C = 19
_NC = 2
_NW = 32
_S = _B + 8
_TBL = 2 * _C * _S
_R = 4


def _sc_body(x_hbm, t_hbm, out_hbm, xs0, ts0, xs1, ts1, sem0, sem1, table):
    n_img, c_img, h_img, w_img = x_hbm.shape
    nvec = (_R * w_img) // 16
    kpr = w_img // 16
    hb_per_img = h_img // _R
    per_w = (n_img * hb_per_img) // _NW
    nhalf = per_w // 2

    wid = lax.axis_index("s") * _NC + lax.axis_index("c")

    def zero(i, carry):
        table[pl.ds(i * 16, 16)] = jnp.zeros((16,), jnp.float32)
        return carry

    lax.fori_loop(0, _TBL // 16, zero, 0)

    ones = jnp.ones((16,), jnp.float32)

    def start_chunk(cid, xs, ts, sem):
        n = cid // hb_per_img
        h0 = (cid % hb_per_img) * _R
        for c in range(_C):
            pltpu.async_copy(x_hbm.at[n, c, pl.ds(h0, _R), :], xs.at[c], sem)
        pltpu.async_copy(t_hbm.at[n, pl.ds(h0, _R), :], ts, sem)

    def drain_chunk(xs, ts, sem):
        for c in range(_C):
            pltpu.make_async_copy(
                x_hbm.at[0, c, pl.ds(0, _R), :], xs.at[c], sem).wait()
        pltpu.make_async_copy(t_hbm.at[0, pl.ds(0, _R), :], ts, sem).wait()

    def process(xs, ts):
        lanes = lax.iota(jnp.int32, 16)

        @plsc.parallel_loop(0, nvec, 1, unroll=1)
        def vb(j):
            r = j // kpr
            k = (j % kpr) * 16
            t = ts[r, pl.ds(k, 16)]
            es = []
            s = None
            for c in range(_C):
                e = jnp.exp(xs[c, r, pl.ds(k, 16)])
                es.append(e)
                s = e if s is None else s + e
            rs = 1.0 / s
            for c in range(_C):
                p = es[c] * rs
                isfg = t == c
                e = jnp.where(isfg, 1.0 - p, p)
                b = (e * _B).astype(jnp.int32)
                plsc.addupdate_scatter(table, [b + (c * _S)], ones)
            xt = plsc.load_gather(xs, [t, jnp.full((16,), r, jnp.int32),
                                       k + lanes])
            pt = jnp.exp(xt) * rs
            bfg = ((1.0 - pt) * _B).astype(jnp.int32)
            plsc.addupdate_scatter(table, [bfg + t * _S + (_C * _S)], ones)

    base = wid * per_w
    start_chunk(base, xs0, ts0, sem0)

    def pair(i, carry):
        c0 = base + 2 * i
        start_chunk(c0 + 1, xs1, ts1, sem1)
        drain_chunk(xs0, ts0, sem0)
        process(xs0, ts0)

        @pl.when(i + 1 < nhalf)
        def _():
            start_chunk(c0 + 2, xs0, ts0, sem0)

        drain_chunk(xs1, ts1, sem1)
        process(xs1, ts1)
        return carry

    lax.fori_loop(0, nhalf, pair, 0)
    pltpu.sync_copy(table, out_hbm.at[wid])


def _histogram(x4, t3):
    mesh = plsc.VectorSubcoreMesh(core_axis_name="c", subcore_axis_name="s")
    return pl.kernel(
        _sc_body,
        out_type=jax.ShapeDtypeStruct((_NW, _TBL), jnp.float32),
        mesh=mesh,
        scratch_types=[
            pltpu.VMEM((_C, _R, x4.shape[3]), jnp.float32),
            pltpu.VMEM((_R, x4.shape[3]), jnp.int32),
            pltpu.VMEM((_C, _R, x4.shape[3]), jnp.float32),
            pltpu.VMEM((_R, x4.shape[3]), jnp.int32),
            pltpu.SemaphoreType.DMA,
            pltpu.SemaphoreType.DMA,
            pltpu.VMEM((_TBL,), jnp.float32),
        ],
        compiler_params=pltpu.CompilerParams(needs_layout_passes=False),
    )(x4, t3)


def _post_body(h_ref, tri_ref, out_ref):
    h = jnp.sum(h_ref[...], axis=0)
    h = jnp.concatenate([h[:, :_B - 1], h[:, _B - 1:_B] + h[:, _B:_B + 1]],
                        axis=1)
    cnt = h[:_C]
    fgc = h[_C:]
    tri = tri_ref[...]
    r = lax.dot(cnt, tri, precision=lax.Precision.HIGHEST)
    f = lax.dot(fgc, tri, precision=lax.Precision.HIGHEST)
    g = f[:, 0:1]
    u = g + r - f
    jac = 1.0 - (g - f) / jnp.maximum(u, 1.0)
    jnext = jnp.concatenate([jac[:, 1:], jnp.zeros((_C, 1), jnp.float32)],
                            axis=1)
    dj = jac - jnext
    mid = ((lax.broadcasted_iota(jnp.int32, (_C, _B), 1)
            .astype(jnp.float32)) + 0.5) / _B
    losses = jnp.sum(mid * dj, axis=1)
    present = (g[:, 0] > 0.0).astype(jnp.float32)
    loss = jnp.sum(losses * present) / jnp.maximum(jnp.sum(present), 1.0)
    out_ref[...] = jnp.reshape(loss, (1, 1))


def _post(hist3, tri):
    return pl.pallas_call(
        _post_body,
        out_shape=jax.ShapeDtypeStruct((1, 1), jnp.float32),
    )(hist3, tri)


def kernel(input, target):
    hist = _histogram(input, target.astype(jnp.int32))
    hist3 = hist.reshape(_NW, 2 * _C, _S)
    tri = jnp.tri(_B, dtype=jnp.float32)
    out = _post(hist3, tri)
    return out[0, 0]
